# baseline (speedup 1.0000x reference)
import functools

import jax
import jax.numpy as jnp
from jax import lax
from jax.experimental import pallas as pl
from jax.experimental.pallas import tpu as pltpu
from jax.experimental.pallas import tpu_sc as plsc

L_HEADS = 16
K_SEL = 2
NEG_INF = float("-inf")


def _logits_kernel(x_ref, w_ref, lt_ref):
    lt_ref[...] = jax.lax.dot_general(w_ref[...], x_ref[...],
                                      (((1,), (1,)), ((), ())),
                                      preferred_element_type=jnp.float32)


def _lane_total(v, iota):
    for step in (8, 4, 2, 1):
        v = v + jnp.take(v, iota ^ step)
    return v


def _sc_router(*refs, tok_per_w, inv_total, emit_loss):
    if emit_loss:
        (lt_hbm, bias_hbm, prev_cnts_hbm, heads_hbm, probs_hbm, cnts_hbm,
         loss_hbm, lt_v, bias_v, heads_v, probs_v, cnt_v, red_v, loss_v) = refs
    else:
        (lt_hbm, bias_hbm, heads_hbm, probs_hbm, cnts_hbm,
         lt_v, bias_v, heads_v, probs_v, cnt_v) = refs
    L = L_HEADS
    sid = lax.axis_index("s")
    base = sid * tok_per_w
    n_groups = tok_per_w // 16

    pltpu.sync_copy(lt_hbm.at[:, pl.ds(base, tok_per_w)], lt_v)
    pltpu.sync_copy(bias_hbm, bias_v)
    iota = lax.iota(jnp.int32, 16)
    one = jnp.full((16,), 1.0, jnp.float32)
    zero = jnp.zeros((16,), jnp.float32)
    bv = bias_v[...]
    bias_b = [jnp.full((16,), bv[l]) for l in range(L)]

    def group(g, accs):
        off = g * 16
        vs = [lt_v[l, pl.ds(off, 16)] for l in range(L)]
        m1 = vs[0] + bias_b[0]
        u1 = vs[0]
        i1 = jnp.zeros((16,), jnp.int32)
        m2 = jnp.full((16,), NEG_INF, jnp.float32)
        u2 = jnp.zeros((16,), jnp.float32)
        i2 = jnp.zeros((16,), jnp.int32)
        for l in range(1, L):
            bl = vs[l] + bias_b[l]
            gt1 = bl > m1
            gt2 = bl > m2
            li = jnp.full((16,), l, jnp.int32)
            m2 = jnp.where(gt1, m1, jnp.where(gt2, bl, m2))
            u2 = jnp.where(gt1, u1, jnp.where(gt2, vs[l], u2))
            i2 = jnp.where(gt1, i1, jnp.where(gt2, li, i2))
            m1 = jnp.where(gt1, bl, m1)
            u1 = jnp.where(gt1, vs[l], u1)
            i1 = jnp.where(gt1, li, i1)
        r = jnp.exp(u2 - u1)
        inv = 1.0 / (1.0 + r)
        heads_v[0, pl.ds(off, 16)] = i1
        heads_v[1, pl.ds(off, 16)] = i2
        probs_v[0, pl.ds(off, 16)] = inv
        probs_v[1, pl.ds(off, 16)] = r * inv
        return [accs[l]
                + jnp.where(i1 == l, one, zero)
                + jnp.where(i2 == l, one, zero)
                for l in range(L)]

    accs = [jnp.zeros((16,), jnp.float32) for _ in range(L)]
    accs = lax.fori_loop(0, n_groups, group, accs)

    cnt = jnp.zeros((16,), jnp.float32)
    for l in range(L):
        tot = _lane_total(accs[l], iota)
        cnt = jnp.where(iota == l, tot, cnt)
    cnt_v[...] = cnt

    pltpu.sync_copy(heads_v, heads_hbm.at[:, pl.ds(base, tok_per_w)])
    pltpu.sync_copy(probs_v, probs_hbm.at[:, pl.ds(base, tok_per_w)])

    pltpu.sync_copy(cnt_v, cnts_hbm.at[sid])

    if emit_loss:
        plsc.subcore_barrier()

        @pl.when(sid == 0)
        def _():
            pltpu.sync_copy(cnts_hbm, red_v)
            tot = red_v[0, :]
            for s in range(1, 16):
                tot = tot + red_v[s, :]
            pltpu.sync_copy(prev_cnts_hbm, red_v)
            for s in range(16):
                tot = tot + red_v[s, :]
            freqs = tot * inv_total
            d = freqs - 1.0 / L
            loss = _lane_total(d * d, iota)
            loss_v[...] = jnp.where(iota == 0, loss, zero)
            pltpu.sync_copy(loss_v, loss_hbm)


def kernel(x, W, expert_bias):
    B, N, H = x.shape
    L = W.shape[1]
    K = K_SEL
    tokens = B * N
    half = tokens // 2
    T = 1024
    n_steps = half // T

    xf = x.reshape(tokens, H)
    wt = W.T

    tc = pl.pallas_call(
        _logits_kernel,
        grid=(n_steps,),
        in_specs=[
            pl.BlockSpec((T, H), lambda i: (i, 0)),
            pl.BlockSpec((L, H), lambda i: (0, 0)),
        ],
        out_specs=pl.BlockSpec((L, T), lambda i: (0, i)),
        out_shape=jax.ShapeDtypeStruct((L, half), jnp.float32),
    )
    lt0 = tc(xf[:half], wt)
    lt1 = tc(xf[half:], wt)

    n_workers = 16
    tok_per_w = half // n_workers
    mesh = plsc.VectorSubcoreMesh(core_axis_name="c", subcore_axis_name="s",
                                  num_cores=1)
    common = dict(tok_per_w=tok_per_w, inv_total=1.0 / (tokens * K))
    sc0 = pl.kernel(
        functools.partial(_sc_router, emit_loss=False, **common),
        mesh=mesh,
        out_type=[
            jax.ShapeDtypeStruct((K, half), jnp.int32),
            jax.ShapeDtypeStruct((K, half), jnp.float32),
            jax.ShapeDtypeStruct((16, 16), jnp.float32),
        ],
        scratch_types=[
            pltpu.VMEM((L, tok_per_w), jnp.float32),
            pltpu.VMEM((L,), jnp.float32),
            pltpu.VMEM((K, tok_per_w), jnp.int32),
            pltpu.VMEM((K, tok_per_w), jnp.float32),
            pltpu.VMEM((16,), jnp.float32),
        ],
    )
    sc1 = pl.kernel(
        functools.partial(_sc_router, emit_loss=True, **common),
        mesh=mesh,
        out_type=[
            jax.ShapeDtypeStruct((K, half), jnp.int32),
            jax.ShapeDtypeStruct((K, half), jnp.float32),
            jax.ShapeDtypeStruct((16, 16), jnp.float32),
            jax.ShapeDtypeStruct((16,), jnp.float32),
        ],
        scratch_types=[
            pltpu.VMEM((L, tok_per_w), jnp.float32),
            pltpu.VMEM((L,), jnp.float32),
            pltpu.VMEM((K, tok_per_w), jnp.int32),
            pltpu.VMEM((K, tok_per_w), jnp.float32),
            pltpu.VMEM((16,), jnp.float32),
            pltpu.VMEM((16, 16), jnp.float32),
            pltpu.VMEM((16,), jnp.float32),
        ],
    )
    heads0, probs0, cnts0 = sc0(lt0, expert_bias)
    heads1, probs1, cnts1, loss_vec = sc1(lt1, expert_bias, cnts0)

    heads = jnp.concatenate([heads0, heads1], axis=1).T.reshape(B, N, K)
    probs = jnp.concatenate([probs0, probs1], axis=1).T.reshape(B, N, K)
    return (heads, probs, loss_vec[0].reshape(()))


# speedup vs baseline: 2.1820x; 2.1820x over previous
import functools

import jax
import jax.numpy as jnp
from jax import lax
from jax.experimental import pallas as pl
from jax.experimental.pallas import tpu as pltpu
from jax.experimental.pallas import tpu_sc as plsc

L_HEADS = 16
K_SEL = 2
NEG_INF = float("-inf")


def _logits_kernel(x_ref, w_ref, lt_ref):
    lt_ref[...] = jax.lax.dot_general(w_ref[...], x_ref[...],
                                      (((1,), (1,)), ((), ())),
                                      preferred_element_type=jnp.float32)


def _lane_total(v, iota):
    for step in (8, 4, 2, 1):
        v = v + jnp.take(v, iota ^ step)
    return v


def _sc_router(lt_hbm, bias_hbm, heads_hbm, probs_hbm, loss_hbm, cnts_hbm,
               lt_v, bias_v, heads_v, probs_v, cnt_v, red_v, loss_v,
               *, tok_per_w, inv_total):
    L = L_HEADS
    sid = lax.axis_index("s")
    base = sid * tok_per_w
    n_groups = tok_per_w // 16

    pltpu.sync_copy(lt_hbm.at[:, pl.ds(base, tok_per_w)], lt_v)
    pltpu.sync_copy(bias_hbm, bias_v)
    iota = lax.iota(jnp.int32, 16)
    one = jnp.full((16,), 1.0, jnp.float32)
    zero = jnp.zeros((16,), jnp.float32)
    bv = bias_v[...]
    bias_b = [jnp.full((16,), bv[l]) for l in range(L)]

    def group(g, accs):
        off = g * 16
        vs = [lt_v[l, pl.ds(off, 16)] for l in range(L)]
        m1 = vs[0] + bias_b[0]
        u1 = vs[0]
        i1 = jnp.zeros((16,), jnp.int32)
        m2 = jnp.full((16,), NEG_INF, jnp.float32)
        u2 = jnp.zeros((16,), jnp.float32)
        i2 = jnp.zeros((16,), jnp.int32)
        for l in range(1, L):
            bl = vs[l] + bias_b[l]
            gt1 = bl > m1
            gt2 = bl > m2
            li = jnp.full((16,), l, jnp.int32)
            m2 = jnp.where(gt1, m1, jnp.where(gt2, bl, m2))
            u2 = jnp.where(gt1, u1, jnp.where(gt2, vs[l], u2))
            i2 = jnp.where(gt1, i1, jnp.where(gt2, li, i2))
            m1 = jnp.where(gt1, bl, m1)
            u1 = jnp.where(gt1, vs[l], u1)
            i1 = jnp.where(gt1, li, i1)
        r = jnp.exp(u2 - u1)
        inv = 1.0 / (1.0 + r)
        heads_v[0, pl.ds(off, 16)] = i1
        heads_v[1, pl.ds(off, 16)] = i2
        probs_v[0, pl.ds(off, 16)] = inv
        probs_v[1, pl.ds(off, 16)] = r * inv
        return [accs[l]
                + jnp.where(i1 == l, one, zero)
                + jnp.where(i2 == l, one, zero)
                for l in range(L)]

    accs = [jnp.zeros((16,), jnp.float32) for _ in range(L)]
    accs = lax.fori_loop(0, n_groups, group, accs)

    cnt = jnp.zeros((16,), jnp.float32)
    for l in range(L):
        tot = _lane_total(accs[l], iota)
        cnt = jnp.where(iota == l, tot, cnt)
    cnt_v[...] = cnt

    pltpu.sync_copy(heads_v, heads_hbm.at[:, pl.ds(base, tok_per_w)])
    pltpu.sync_copy(probs_v, probs_hbm.at[:, pl.ds(base, tok_per_w)])

    pltpu.sync_copy(cnt_v, cnts_hbm.at[sid])
    plsc.subcore_barrier()

    @pl.when(sid == 0)
    def _():
        pltpu.sync_copy(cnts_hbm, red_v)
        tot = red_v[0, :]
        for s in range(1, 16):
            tot = tot + red_v[s, :]
        freqs = tot * inv_total
        d = freqs - 1.0 / L
        loss = _lane_total(d * d, iota)
        loss_v[...] = jnp.where(iota == 0, loss, zero)
        pltpu.sync_copy(loss_v, loss_hbm)


def kernel(x, W, expert_bias):
    B, N, H = x.shape
    L = W.shape[1]
    K = K_SEL
    tokens = B * N
    T = 1024
    n_steps = tokens // T

    xf = x.reshape(tokens, H)
    wt = W.T

    lt = pl.pallas_call(
        _logits_kernel,
        grid=(n_steps,),
        in_specs=[
            pl.BlockSpec((T, H), lambda i: (i, 0)),
            pl.BlockSpec((L, H), lambda i: (0, 0)),
        ],
        out_specs=pl.BlockSpec((L, T), lambda i: (0, i)),
        out_shape=jax.ShapeDtypeStruct((L, tokens), jnp.float32),
    )(xf, wt)

    n_workers = 16
    tok_per_w = tokens // n_workers
    mesh = plsc.VectorSubcoreMesh(core_axis_name="c", subcore_axis_name="s",
                                  num_cores=1)
    sc = pl.kernel(
        functools.partial(_sc_router, tok_per_w=tok_per_w,
                          inv_total=1.0 / (tokens * K)),
        mesh=mesh,
        out_type=[
            jax.ShapeDtypeStruct((K, tokens), jnp.int32),
            jax.ShapeDtypeStruct((K, tokens), jnp.float32),
            jax.ShapeDtypeStruct((16,), jnp.float32),
            jax.ShapeDtypeStruct((16, 16), jnp.float32),
        ],
        scratch_types=[
            pltpu.VMEM((L, tok_per_w), jnp.float32),
            pltpu.VMEM((L,), jnp.float32),
            pltpu.VMEM((K, tok_per_w), jnp.int32),
            pltpu.VMEM((K, tok_per_w), jnp.float32),
            pltpu.VMEM((16,), jnp.float32),
            pltpu.VMEM((16, 16), jnp.float32),
            pltpu.VMEM((16,), jnp.float32),
        ],
    )
    heads_t, probs_t, loss_vec, _cnts = sc(lt, expert_bias)

    heads = heads_t.T.reshape(B, N, K)
    probs = probs_t.T.reshape(B, N, K)
    return (heads, probs, loss_vec[0].reshape(()))


# speedup vs baseline: 2.1859x; 1.0018x over previous
import functools

import jax
import jax.numpy as jnp
from jax import lax
from jax.experimental import pallas as pl
from jax.experimental.pallas import tpu as pltpu
from jax.experimental.pallas import tpu_sc as plsc

L_HEADS = 16
K_SEL = 2
NEG_INF = float("-inf")


def _logits_kernel(x_ref, w_ref, lt_ref):
    lt_ref[...] = jax.lax.dot_general(w_ref[...], x_ref[...],
                                      (((1,), (1,)), ((), ())),
                                      preferred_element_type=jnp.float32)


def _lane_total(v, iota):
    for step in (8, 4, 2, 1):
        v = v + jnp.take(v, iota ^ step)
    return v


def _sc_router(lt_hbm, bias_hbm, heads_hbm, probs_hbm, loss_hbm, cnts_hbm,
               lt_v, bias_v, heads_v, probs_v, cnt_v, red_v, loss_v,
               *, tok_per_w, inv_total):
    L = L_HEADS
    sid = lax.axis_index("s")
    base = sid * tok_per_w
    n_groups = tok_per_w // 16

    pltpu.sync_copy(lt_hbm.at[:, pl.ds(base, tok_per_w)], lt_v)
    pltpu.sync_copy(bias_hbm, bias_v)
    iota = lax.iota(jnp.int32, 16)
    one = jnp.full((16,), 1.0, jnp.float32)
    zero = jnp.zeros((16,), jnp.float32)
    bv = bias_v[...]
    bias_b = [jnp.full((16,), bv[l]) for l in range(L)]

    def group(off, accs):
        vs = [lt_v[l, pl.ds(off, 16)] for l in range(L)]
        m1 = vs[0] + bias_b[0]
        u1 = vs[0]
        i1 = jnp.zeros((16,), jnp.int32)
        m2 = jnp.full((16,), NEG_INF, jnp.float32)
        u2 = jnp.zeros((16,), jnp.float32)
        i2 = jnp.zeros((16,), jnp.int32)
        for l in range(1, L):
            bl = vs[l] + bias_b[l]
            gt1 = bl > m1
            gt2 = bl > m2
            li = jnp.full((16,), l, jnp.int32)
            m2 = jnp.where(gt1, m1, jnp.where(gt2, bl, m2))
            u2 = jnp.where(gt1, u1, jnp.where(gt2, vs[l], u2))
            i2 = jnp.where(gt1, i1, jnp.where(gt2, li, i2))
            m1 = jnp.where(gt1, bl, m1)
            u1 = jnp.where(gt1, vs[l], u1)
            i1 = jnp.where(gt1, li, i1)
        r = jnp.exp(u2 - u1)
        inv = 1.0 / (1.0 + r)
        heads_v[0, pl.ds(off, 16)] = i1
        heads_v[1, pl.ds(off, 16)] = i2
        probs_v[0, pl.ds(off, 16)] = inv
        probs_v[1, pl.ds(off, 16)] = r * inv
        return [accs[l]
                + jnp.where(i1 == l, one, zero)
                + jnp.where(i2 == l, one, zero)
                for l in range(L)]

    accs = [jnp.zeros((16,), jnp.float32) for _ in range(L)]
    accs = plsc.parallel_loop(0, n_groups * 16, step=16, carry=accs)(group)

    cnt = jnp.zeros((16,), jnp.float32)
    for l in range(L):
        tot = _lane_total(accs[l], iota)
        cnt = jnp.where(iota == l, tot, cnt)
    cnt_v[...] = cnt

    pltpu.sync_copy(heads_v, heads_hbm.at[:, pl.ds(base, tok_per_w)])
    pltpu.sync_copy(probs_v, probs_hbm.at[:, pl.ds(base, tok_per_w)])

    pltpu.sync_copy(cnt_v, cnts_hbm.at[sid])
    plsc.subcore_barrier()

    @pl.when(sid == 0)
    def _():
        pltpu.sync_copy(cnts_hbm, red_v)
        tot = red_v[0, :]
        for s in range(1, 16):
            tot = tot + red_v[s, :]
        freqs = tot * inv_total
        d = freqs - 1.0 / L
        loss = _lane_total(d * d, iota)
        loss_v[...] = jnp.where(iota == 0, loss, zero)
        pltpu.sync_copy(loss_v, loss_hbm)


def kernel(x, W, expert_bias):
    B, N, H = x.shape
    L = W.shape[1]
    K = K_SEL
    tokens = B * N
    T = 1024
    n_steps = tokens // T

    xf = x.reshape(tokens, H)
    wt = W.T

    lt = pl.pallas_call(
        _logits_kernel,
        grid=(n_steps,),
        in_specs=[
            pl.BlockSpec((T, H), lambda i: (i, 0)),
            pl.BlockSpec((L, H), lambda i: (0, 0)),
        ],
        out_specs=pl.BlockSpec((L, T), lambda i: (0, i)),
        out_shape=jax.ShapeDtypeStruct((L, tokens), jnp.float32),
    )(xf, wt)

    n_workers = 16
    tok_per_w = tokens // n_workers
    mesh = plsc.VectorSubcoreMesh(core_axis_name="c", subcore_axis_name="s",
                                  num_cores=1)
    sc = pl.kernel(
        functools.partial(_sc_router, tok_per_w=tok_per_w,
                          inv_total=1.0 / (tokens * K)),
        mesh=mesh,
        out_type=[
            jax.ShapeDtypeStruct((K, tokens), jnp.int32),
            jax.ShapeDtypeStruct((K, tokens), jnp.float32),
            jax.ShapeDtypeStruct((16,), jnp.float32),
            jax.ShapeDtypeStruct((16, 16), jnp.float32),
        ],
        scratch_types=[
            pltpu.VMEM((L, tok_per_w), jnp.float32),
            pltpu.VMEM((L,), jnp.float32),
            pltpu.VMEM((K, tok_per_w), jnp.int32),
            pltpu.VMEM((K, tok_per_w), jnp.float32),
            pltpu.VMEM((16,), jnp.float32),
            pltpu.VMEM((16, 16), jnp.float32),
            pltpu.VMEM((16,), jnp.float32),
        ],
    )
    heads_t, probs_t, loss_vec, _cnts = sc(lt, expert_bias)

    heads = heads_t.T.reshape(B, N, K)
    probs = probs_t.T.reshape(B, N, K)
    return (heads, probs, loss_vec[0].reshape(()))
